# baseline (speedup 1.0000x reference)
import functools

import jax
import jax.numpy as jnp
from jax import lax
from jax.experimental import pallas as pl
from jax.experimental.pallas import tpu as pltpu
from jax.experimental.pallas import tpu_sc as plsc

_NC = 2
_NS = 16
_NW = _NC * _NS
_CHUNK = 16


_NBUF = 6
_HD = 8


def _gather_body(tt_hbm, idx_hbm, xt_hbm, idx_v, slab_v, xt_v, sem):
    d = tt_hbm.shape[0]
    b_per_w = idx_v.shape[0]
    ngrp = b_per_w // _CHUNK
    wid = lax.axis_index("s") * _NC + lax.axis_index("c")
    base = wid * b_per_w
    pltpu.sync_copy(idx_hbm.at[pl.ds(base, b_per_w)], idx_v)

    iota = lax.iota(jnp.int32, 16)

    for h in range(d // _HD):
        r0 = h * _HD

        def fire(g, h=h, r0=r0):
            buf = g % _NBUF
            vblk = (idx_v[pl.ds(g * _CHUNK, 16)] // 128) * 128
            for t in range(_CHUNK):
                blk = pl.multiple_of(vblk[t], 128)
                pltpu.async_copy(
                    tt_hbm.at[pl.ds(r0, _HD), pl.ds(blk, 128)],
                    slab_v.at[buf, t],
                    sem,
                )

        def drain_one(g):
            buf = g % _NBUF
            for t in range(_CHUNK):
                pltpu.make_async_copy(
                    tt_hbm.at[pl.ds(0, _HD), pl.ds(0, 128)],
                    slab_v.at[buf, t],
                    sem,
                ).wait()

        fire(0)
        fire(1)
        fire(2)
        fire(3)

        def group_body(g, carry, h=h, r0=r0):
            @pl.when(g + 4 < ngrp)
            def _():
                fire(g + 4)

            drain_one(g)
            bufv = jnp.full((16,), 0, jnp.int32) + (g % _NBUF)
            rem = idx_v[pl.ds(g * _CHUNK, 16)] & 127
            for cc in range(_HD):
                ccv = jnp.full((16,), cc, jnp.int32)
                xt_v[r0 + cc, pl.ds(g * _CHUNK, 16)] = plsc.load_gather(
                    slab_v, [bufv, iota, ccv, rem]
                )
            return carry

        lax.fori_loop(0, ngrp, group_body, 0)

    pltpu.sync_copy(xt_v, xt_hbm.at[:, pl.ds(base, b_per_w)])


@functools.partial(jax.jit, static_argnames=("b", "d"))
def _sc_gather_t(tt, idx, b, d):
    b_per_w = b // _NW
    mesh = plsc.VectorSubcoreMesh(
        core_axis_name="c", subcore_axis_name="s", num_cores=_NC,
        num_subcores=_NS,
    )
    return pl.kernel(
        _gather_body,
        out_type=jax.ShapeDtypeStruct((d, b), jnp.float32),
        mesh=mesh,
        scratch_types=[
            pltpu.VMEM((b_per_w,), jnp.int32),
            pltpu.VMEM((_NBUF, _CHUNK, _HD, 128), jnp.float32),
            pltpu.VMEM((d, b_per_w), jnp.float32),
            pltpu.SemaphoreType.DMA,
        ],
        compiler_params=pltpu.CompilerParams(needs_layout_passes=False),
    )(tt, idx)


def _cov_body(xa_ref, xb_ref, o_ref):
    o_ref[...] = lax.dot_general(
        xa_ref[...], xb_ref[...],
        dimension_numbers=(((0,), (0,)), ((), ())),
        preferred_element_type=jnp.float32,
    )


_BM = 512


def _tc_cov_t(xt):
    d, b = xt.shape
    return pl.pallas_call(
        _cov_body,
        grid=(b // _BM,),
        in_specs=[
            pl.BlockSpec((d, _BM), lambda i: (0, i)),
            pl.BlockSpec((d, b), lambda i: (0, 0)),
        ],
        out_specs=pl.BlockSpec((_BM, b), lambda i: (i, 0)),
        out_shape=jax.ShapeDtypeStruct((b, b), jnp.float32),
        compiler_params=pltpu.CompilerParams(
            dimension_semantics=("arbitrary",),
        ),
    )(xt, xt)


def kernel(states, table):
    b = states.shape[0]
    d = table.shape[1]
    idx = states.reshape(b).astype(jnp.int32)
    xt = _sc_gather_t(table.T, idx, b, d)
    cov = _tc_cov_t(xt)
    return (xt.T, cov)


# speedup vs baseline: 1.0556x; 1.0556x over previous
import functools

import jax
import jax.numpy as jnp
from jax import lax
from jax.experimental import pallas as pl
from jax.experimental.pallas import tpu as pltpu
from jax.experimental.pallas import tpu_sc as plsc

_NC = 2
_NS = 16
_NW = _NC * _NS
_CHUNK = 16


_NBUF = 3
_HD = 16


def _gather_body(tt_hbm, idx_hbm, xt_hbm, idx_v, slab_v, xt_v, sem):
    d = tt_hbm.shape[0]
    b_per_w = idx_v.shape[0]
    ngrp = b_per_w // _CHUNK
    wid = lax.axis_index("s") * _NC + lax.axis_index("c")
    base = wid * b_per_w
    pltpu.sync_copy(idx_hbm.at[pl.ds(base, b_per_w)], idx_v)

    iota = lax.iota(jnp.int32, 16)

    for h in range(d // _HD):
        r0 = h * _HD

        def fire(g, h=h, r0=r0):
            buf = g % _NBUF
            vblk = (idx_v[pl.ds(g * _CHUNK, 16)] // 128) * 128
            for t in range(_CHUNK):
                blk = pl.multiple_of(vblk[t], 128)
                pltpu.async_copy(
                    tt_hbm.at[pl.ds(r0, _HD), pl.ds(blk, 128)],
                    slab_v.at[buf, t],
                    sem,
                )

        def drain_one(g):
            buf = g % _NBUF
            for t in range(_CHUNK):
                pltpu.make_async_copy(
                    tt_hbm.at[pl.ds(0, _HD), pl.ds(0, 128)],
                    slab_v.at[buf, t],
                    sem,
                ).wait()

        fire(0)
        fire(1)

        def group_body(g, carry, h=h, r0=r0):
            @pl.when(g + 2 < ngrp)
            def _():
                fire(g + 2)

            drain_one(g)
            bufv = jnp.full((16,), 0, jnp.int32) + (g % _NBUF)
            rem = idx_v[pl.ds(g * _CHUNK, 16)] & 127
            for cc in range(_HD):
                ccv = jnp.full((16,), cc, jnp.int32)
                xt_v[r0 + cc, pl.ds(g * _CHUNK, 16)] = plsc.load_gather(
                    slab_v, [bufv, iota, ccv, rem]
                )
            return carry

        lax.fori_loop(0, ngrp, group_body, 0)

    pltpu.sync_copy(xt_v, xt_hbm.at[:, pl.ds(base, b_per_w)])


@functools.partial(jax.jit, static_argnames=("b", "d"))
def _sc_gather_t(tt, idx, b, d):
    b_per_w = b // _NW
    mesh = plsc.VectorSubcoreMesh(
        core_axis_name="c", subcore_axis_name="s", num_cores=_NC,
        num_subcores=_NS,
    )
    return pl.kernel(
        _gather_body,
        out_type=jax.ShapeDtypeStruct((d, b), jnp.float32),
        mesh=mesh,
        scratch_types=[
            pltpu.VMEM((b_per_w,), jnp.int32),
            pltpu.VMEM((_NBUF, _CHUNK, _HD, 128), jnp.float32),
            pltpu.VMEM((d, b_per_w), jnp.float32),
            pltpu.SemaphoreType.DMA,
        ],
        compiler_params=pltpu.CompilerParams(needs_layout_passes=False),
    )(tt, idx)


def _cov_body(xa_ref, xb_ref, o_ref):
    o_ref[...] = lax.dot_general(
        xa_ref[...], xb_ref[...],
        dimension_numbers=(((0,), (0,)), ((), ())),
        preferred_element_type=jnp.float32,
    )


_BM = 512


def _tc_cov_t(xt):
    d, b = xt.shape
    return pl.pallas_call(
        _cov_body,
        grid=(b // _BM,),
        in_specs=[
            pl.BlockSpec((d, _BM), lambda i: (0, i)),
            pl.BlockSpec((d, b), lambda i: (0, 0)),
        ],
        out_specs=pl.BlockSpec((_BM, b), lambda i: (i, 0)),
        out_shape=jax.ShapeDtypeStruct((b, b), jnp.float32),
        compiler_params=pltpu.CompilerParams(
            dimension_semantics=("arbitrary",),
        ),
    )(xt, xt)


def kernel(states, table):
    b = states.shape[0]
    d = table.shape[1]
    idx = states.reshape(b).astype(jnp.int32)
    xt = _sc_gather_t(table.T, idx, b, d)
    cov = _tc_cov_t(xt)
    return (xt.T, cov)
